# baseline (speedup 1.0000x reference)
import functools

import jax
import jax.numpy as jnp
from jax import lax
from jax.experimental import pallas as pl
from jax.experimental.pallas import tpu as pltpu
from jax.experimental.pallas import tpu_sc as plsc

F = 128
FW = F + 2
NC, NS = 2, 16
NW = NC * NS
L = 16
CH = 80


def _tc_prep(xr, w_s, n_pad):
    n = xr.shape[0]
    bm = 2000 if n % 2000 == 0 else 1000

    def body(in_ref, ws_ref, x_ref, y_ref, safe_ref, val_ref):
        blk = in_ref[...]
        x = blk[:, :F]
        x_ref[...] = x
        y_ref[...] = jnp.dot(x, ws_ref[...], preferred_element_type=jnp.float32)
        pair = blk[:, F:FW].astype(jnp.int32)
        valid = (pair > 0) & (pair < n)
        rows = pl.program_id(0) * bm + lax.broadcasted_iota(
            jnp.int32, (bm, 2), 0)
        safe_ref[...] = jnp.where(valid, pair, rows)
        val_ref[...] = jnp.where(valid, 1.0, 0.0)

    return pl.pallas_call(
        body,
        grid=(n // bm,),
        in_specs=[
            pl.BlockSpec((bm, FW), lambda i: (i, 0)),
            pl.BlockSpec((F, F), lambda i: (0, 0)),
        ],
        out_specs=[
            pl.BlockSpec((bm, F), lambda i: (i, 0)),
            pl.BlockSpec((bm, F), lambda i: (i, 0)),
            pl.BlockSpec((bm, 2), lambda i: (i, 0)),
            pl.BlockSpec((bm, 2), lambda i: (i, 0)),
        ],
        out_shape=[
            jax.ShapeDtypeStruct((n, F), jnp.float32),
            jax.ShapeDtypeStruct((n, F), jnp.float32),
            jax.ShapeDtypeStruct((n_pad, 2), jnp.int32),
            jax.ShapeDtypeStruct((n_pad, 2), jnp.float32),
        ],
    )(xr, w_s)


def _sc_gather_scale(x, s0a, s1a, v0a, v1a, n_pad):
    n = x.shape[0]
    bw = n_pad // NW

    mesh = plsc.VectorSubcoreMesh(core_axis_name="c", subcore_axis_name="s")

    @functools.partial(
        pl.kernel,
        out_type=jax.ShapeDtypeStruct((n_pad, F), jnp.float32),
        mesh=mesh,
        compiler_params=pltpu.CompilerParams(needs_layout_passes=False),
        scratch_types=[
            pltpu.VMEM((bw,), jnp.int32),
            pltpu.VMEM((bw,), jnp.int32),
            pltpu.VMEM((bw,), jnp.float32),
            pltpu.VMEM((bw,), jnp.float32),
            pltpu.VMEM((CH, F), jnp.float32),
            pltpu.VMEM((CH, F), jnp.float32),
            pltpu.VMEM((CH, F), jnp.float32),
            pltpu.VMEM((L, L), jnp.float32),
            pltpu.VMEM((L, L), jnp.float32),
            pltpu.VMEM((CH,), jnp.float32),
            pltpu.VMEM((CH,), jnp.float32),
            pltpu.VMEM_SHARED((n, F), jnp.float32),
            pltpu.SemaphoreType.DMA,
            pltpu.SemaphoreType.DMA,
            pltpu.SemaphoreType.DMA,
            pltpu.SemaphoreType.DMA,
        ],
    )
    def k(x_hbm, s0_hbm, s1_hbm, v0_hbm, v1_hbm, g_hbm,
          safe0, safe1, val0, val1, selfv, nb0, nb1, tr0, tr1,
          cbuf0, cbuf1, x_sh, sem_s, sem0, sem1, sem_sh):
        sid = lax.axis_index("s")
        wid = sid * NC + lax.axis_index("c")
        base = wid * bw

        rps = (n // NS) // 8 * 8
        rem = n - (NS - 1) * rps

        @pl.when(sid < NS - 1)
        def _():
            pltpu.async_copy(
                x_hbm.at[pl.ds(sid * rps, rps)],
                x_sh.at[pl.ds(sid * rps, rps)],
                sem_sh,
            )

        @pl.when(sid == NS - 1)
        def _():
            pltpu.async_copy(
                x_hbm.at[pl.ds((NS - 1) * rps, rem)],
                x_sh.at[pl.ds((NS - 1) * rps, rem)],
                sem_sh,
            )

        lane = lax.iota(jnp.int32, L)
        keep = lane >= 3

        pltpu.sync_copy(s0_hbm.at[pl.ds(base, bw)], safe0)
        pltpu.sync_copy(s1_hbm.at[pl.ds(base, bw)], safe1)
        pltpu.sync_copy(v0_hbm.at[pl.ds(base, bw)], val0)
        pltpu.sync_copy(v1_hbm.at[pl.ds(base, bw)], val1)

        @pl.when(sid < NS - 1)
        def _():
            pltpu.make_async_copy(
                x_hbm.at[pl.ds(sid * rps, rps)],
                x_sh.at[pl.ds(sid * rps, rps)],
                sem_sh,
            ).wait()

        @pl.when(sid == NS - 1)
        def _():
            pltpu.make_async_copy(
                x_hbm.at[pl.ds((NS - 1) * rps, rem)],
                x_sh.at[pl.ds((NS - 1) * rps, rem)],
                sem_sh,
            ).wait()

        plsc.subcore_barrier()

        for c in range(bw // CH):
            cbase = c * CH

            @pl.when(base + cbase < n)
            def _():
                cp_self = pltpu.async_copy(
                    x_hbm.at[pl.ds(base + cbase, CH)], selfv, sem_s)
                cp0 = pltpu.async_copy(
                    x_sh.at[safe0.at[pl.ds(cbase, CH)]], nb0, sem0)
                cp1 = pltpu.async_copy(
                    x_sh.at[safe1.at[pl.ds(cbase, CH)]], nb1, sem1)
                cp_self.wait()
                cp0.wait()
                cp1.wait()

                def per_group(j, _):
                    gbase = j * L
                    for t in range(L):
                        a = gbase + t
                        acc0 = jnp.zeros((L,), jnp.float32)
                        acc1 = jnp.zeros((L,), jnp.float32)
                        for b in range(F // L):
                            s = selfv[a, pl.ds(b * L, L)]
                            e0 = s - nb0[a, pl.ds(b * L, L)]
                            e1 = s - nb1[a, pl.ds(b * L, L)]
                            if b == 0:
                                e0 = jnp.where(keep, e0, 0.0)
                                e1 = jnp.where(keep, e1, 0.0)
                            acc0 = acc0 + e0 * e0
                            acc1 = acc1 + e1 * e1
                        col = jnp.full((L,), t, jnp.int32)
                        plsc.store_scatter(tr0, [lane, col], acc0)
                        plsc.store_scatter(tr1, [lane, col], acc1)
                    d20 = jnp.zeros((L,), jnp.float32)
                    d21 = jnp.zeros((L,), jnp.float32)
                    for r in range(L):
                        d20 = d20 + tr0[r, :]
                        d21 = d21 + tr1[r, :]
                    c0 = jnp.where(
                        d20 > 0, jnp.where(d20 > 1e-6, 1.0 / d20, 1e6), 1.0)
                    c1 = jnp.where(
                        d21 > 0, jnp.where(d21 > 1e-6, 1.0 / d21, 1e6), 1.0)
                    cbuf0[pl.ds(gbase, L)] = c0 * val0[pl.ds(cbase + gbase, L)]
                    cbuf1[pl.ds(gbase, L)] = c1 * val1[pl.ds(cbase + gbase, L)]
                    return 0

                lax.fori_loop(0, CH // L, per_group, 0)

                def per_group_out(j, _):
                    gbase = j * L
                    cv0 = cbuf0[pl.ds(gbase, L)]
                    cv1 = cbuf1[pl.ds(gbase, L)]
                    for t in range(L):
                        a = gbase + t
                        c0 = cv0[t]
                        c1 = cv1[t]
                        for b in range(F // L):
                            selfv[a, pl.ds(b * L, L)] = (
                                c0 * nb0[a, pl.ds(b * L, L)]
                                + c1 * nb1[a, pl.ds(b * L, L)]
                            )
                    return 0

                lax.fori_loop(0, CH // L, per_group_out, 0)
                pltpu.sync_copy(selfv, g_hbm.at[pl.ds(base + cbase, CH)])

    return k(x, s0a, s1a, v0a, v1a)


def _tc_finish(y, g, w_n):
    n = y.shape[0]
    bm = 2000 if n % 2000 == 0 else 1000

    def body(y_ref, g_ref, wn_ref, o_ref):
        o_ref[...] = y_ref[...] + jnp.dot(
            g_ref[...], wn_ref[...], preferred_element_type=jnp.float32)

    return pl.pallas_call(
        body,
        grid=(n // bm,),
        in_specs=[
            pl.BlockSpec((bm, F), lambda i: (i, 0)),
            pl.BlockSpec((bm, F), lambda i: (i, 0)),
            pl.BlockSpec((F, F), lambda i: (0, 0)),
        ],
        out_specs=pl.BlockSpec((bm, F), lambda i: (i, 0)),
        out_shape=jax.ShapeDtypeStruct((n, F), jnp.float32),
    )(y, g, w_n)


def kernel(inputs, w_s, w_n):
    n = inputs.shape[1]
    n_pad = -(-n // (NW * CH)) * (NW * CH)
    xr = inputs[0]
    x, y, safe, val = _tc_prep(xr, w_s, n_pad)
    g = _sc_gather_scale(x, safe[:, 0], safe[:, 1], val[:, 0], val[:, 1], n_pad)
    out = _tc_finish(y, g, w_n)
    return out[None]


# speedup vs baseline: 1.0905x; 1.0905x over previous
import functools

import jax
import jax.numpy as jnp
from jax import lax
from jax.experimental import pallas as pl
from jax.experimental.pallas import tpu as pltpu
from jax.experimental.pallas import tpu_sc as plsc

F = 128
FW = F + 2
NC, NS = 2, 16
NW = NC * NS
L = 16
CH = 80


def _tc_prep(inputs, w_s, n_pad):
    n = inputs.shape[1]
    bm = 2048

    def body(in_ref, ws_ref, x_ref, y_ref, s0_ref, s1_ref, v0_ref, v1_ref):
        blk = in_ref[0]
        x = blk[:, :F]
        x_ref[...] = x
        y_ref[...] = jnp.dot(x, ws_ref[...], preferred_element_type=jnp.float32)
        rows = pl.program_id(0) * bm + lax.iota(jnp.int32, bm)
        for col, s_ref, v_ref in ((F, s0_ref, v0_ref), (F + 1, s1_ref, v1_ref)):
            iv = blk[:, col].astype(jnp.int32)
            valid = (iv > 0) & (iv < n)
            s_ref[...] = jnp.where(valid, iv, rows)
            v_ref[...] = jnp.where(valid, 1.0, 0.0)

    return pl.pallas_call(
        body,
        grid=(n_pad // bm,),
        in_specs=[
            pl.BlockSpec((1, bm, FW), lambda i: (0, i, 0)),
            pl.BlockSpec((F, F), lambda i: (0, 0)),
        ],
        out_specs=[
            pl.BlockSpec((bm, F), lambda i: (i, 0)),
            pl.BlockSpec((bm, F), lambda i: (i, 0)),
            pl.BlockSpec((bm,), lambda i: (i,)),
            pl.BlockSpec((bm,), lambda i: (i,)),
            pl.BlockSpec((bm,), lambda i: (i,)),
            pl.BlockSpec((bm,), lambda i: (i,)),
        ],
        out_shape=[
            jax.ShapeDtypeStruct((n, F), jnp.float32),
            jax.ShapeDtypeStruct((n, F), jnp.float32),
            jax.ShapeDtypeStruct((n_pad,), jnp.int32),
            jax.ShapeDtypeStruct((n_pad,), jnp.int32),
            jax.ShapeDtypeStruct((n_pad,), jnp.float32),
            jax.ShapeDtypeStruct((n_pad,), jnp.float32),
        ],
    )(inputs, w_s)


def _sc_gather_scale(x, s0a, s1a, v0a, v1a, n_pad):
    n = x.shape[0]
    bw = n_pad // NW

    mesh = plsc.VectorSubcoreMesh(core_axis_name="c", subcore_axis_name="s")

    @functools.partial(
        pl.kernel,
        out_type=jax.ShapeDtypeStruct((n_pad, F), jnp.float32),
        mesh=mesh,
        compiler_params=pltpu.CompilerParams(needs_layout_passes=False),
        scratch_types=[
            pltpu.VMEM((bw,), jnp.int32),
            pltpu.VMEM((bw,), jnp.int32),
            pltpu.VMEM((bw,), jnp.float32),
            pltpu.VMEM((bw,), jnp.float32),
            pltpu.VMEM((CH, F), jnp.float32),
            pltpu.VMEM((CH, F), jnp.float32),
            pltpu.VMEM((CH, F), jnp.float32),
            pltpu.VMEM((L, L), jnp.float32),
            pltpu.VMEM((L, L), jnp.float32),
            pltpu.VMEM((CH,), jnp.float32),
            pltpu.VMEM((CH,), jnp.float32),
            pltpu.VMEM_SHARED((n, F), jnp.float32),
            pltpu.SemaphoreType.DMA,
            pltpu.SemaphoreType.DMA,
            pltpu.SemaphoreType.DMA,
            pltpu.SemaphoreType.DMA,
        ],
    )
    def k(x_hbm, s0_hbm, s1_hbm, v0_hbm, v1_hbm, g_hbm,
          safe0, safe1, val0, val1, selfv, nb0, nb1, tr0, tr1,
          cbuf0, cbuf1, x_sh, sem_s, sem0, sem1, sem_sh):
        sid = lax.axis_index("s")
        wid = sid * NC + lax.axis_index("c")
        base = wid * bw

        rps = (n // NS) // 8 * 8
        rem = n - (NS - 1) * rps

        @pl.when(sid < NS - 1)
        def _():
            pltpu.async_copy(
                x_hbm.at[pl.ds(sid * rps, rps)],
                x_sh.at[pl.ds(sid * rps, rps)],
                sem_sh,
            )

        @pl.when(sid == NS - 1)
        def _():
            pltpu.async_copy(
                x_hbm.at[pl.ds((NS - 1) * rps, rem)],
                x_sh.at[pl.ds((NS - 1) * rps, rem)],
                sem_sh,
            )

        lane = lax.iota(jnp.int32, L)
        keep = lane >= 3

        pltpu.sync_copy(s0_hbm.at[pl.ds(base, bw)], safe0)
        pltpu.sync_copy(s1_hbm.at[pl.ds(base, bw)], safe1)
        pltpu.sync_copy(v0_hbm.at[pl.ds(base, bw)], val0)
        pltpu.sync_copy(v1_hbm.at[pl.ds(base, bw)], val1)

        @pl.when(sid < NS - 1)
        def _():
            pltpu.make_async_copy(
                x_hbm.at[pl.ds(sid * rps, rps)],
                x_sh.at[pl.ds(sid * rps, rps)],
                sem_sh,
            ).wait()

        @pl.when(sid == NS - 1)
        def _():
            pltpu.make_async_copy(
                x_hbm.at[pl.ds((NS - 1) * rps, rem)],
                x_sh.at[pl.ds((NS - 1) * rps, rem)],
                sem_sh,
            ).wait()

        plsc.subcore_barrier()

        for c in range(bw // CH):
            cbase = c * CH

            @pl.when(base + cbase < n)
            def _():
                cp_self = pltpu.async_copy(
                    x_hbm.at[pl.ds(base + cbase, CH)], selfv, sem_s)
                cp0 = pltpu.async_copy(
                    x_sh.at[safe0.at[pl.ds(cbase, CH)]], nb0, sem0)
                cp1 = pltpu.async_copy(
                    x_sh.at[safe1.at[pl.ds(cbase, CH)]], nb1, sem1)
                cp_self.wait()
                cp0.wait()
                cp1.wait()

                def per_group(j, _):
                    gbase = j * L
                    for t in range(L):
                        a = gbase + t
                        acc0 = jnp.zeros((L,), jnp.float32)
                        acc1 = jnp.zeros((L,), jnp.float32)
                        for b in range(F // L):
                            s = selfv[a, pl.ds(b * L, L)]
                            e0 = s - nb0[a, pl.ds(b * L, L)]
                            e1 = s - nb1[a, pl.ds(b * L, L)]
                            if b == 0:
                                e0 = jnp.where(keep, e0, 0.0)
                                e1 = jnp.where(keep, e1, 0.0)
                            acc0 = acc0 + e0 * e0
                            acc1 = acc1 + e1 * e1
                        col = jnp.full((L,), t, jnp.int32)
                        plsc.store_scatter(tr0, [lane, col], acc0)
                        plsc.store_scatter(tr1, [lane, col], acc1)
                    d20 = jnp.zeros((L,), jnp.float32)
                    d21 = jnp.zeros((L,), jnp.float32)
                    for r in range(L):
                        d20 = d20 + tr0[r, :]
                        d21 = d21 + tr1[r, :]
                    c0 = jnp.where(
                        d20 > 0, jnp.where(d20 > 1e-6, 1.0 / d20, 1e6), 1.0)
                    c1 = jnp.where(
                        d21 > 0, jnp.where(d21 > 1e-6, 1.0 / d21, 1e6), 1.0)
                    cbuf0[pl.ds(gbase, L)] = c0 * val0[pl.ds(cbase + gbase, L)]
                    cbuf1[pl.ds(gbase, L)] = c1 * val1[pl.ds(cbase + gbase, L)]
                    return 0

                lax.fori_loop(0, CH // L, per_group, 0)

                def per_group_out(j, _):
                    gbase = j * L
                    cv0 = cbuf0[pl.ds(gbase, L)]
                    cv1 = cbuf1[pl.ds(gbase, L)]
                    for t in range(L):
                        a = gbase + t
                        c0 = cv0[t]
                        c1 = cv1[t]
                        for b in range(F // L):
                            selfv[a, pl.ds(b * L, L)] = (
                                c0 * nb0[a, pl.ds(b * L, L)]
                                + c1 * nb1[a, pl.ds(b * L, L)]
                            )
                    return 0

                lax.fori_loop(0, CH // L, per_group_out, 0)
                pltpu.sync_copy(selfv, g_hbm.at[pl.ds(base + cbase, CH)])

    return k(x, s0a, s1a, v0a, v1a)


def _tc_finish(y, g, w_n):
    n = y.shape[0]
    bm = 2000 if n % 2000 == 0 else 1000

    def body(y_ref, g_ref, wn_ref, o_ref):
        o_ref[...] = y_ref[...] + jnp.dot(
            g_ref[...], wn_ref[...], preferred_element_type=jnp.float32)

    return pl.pallas_call(
        body,
        grid=(n // bm,),
        in_specs=[
            pl.BlockSpec((bm, F), lambda i: (i, 0)),
            pl.BlockSpec((bm, F), lambda i: (i, 0)),
            pl.BlockSpec((F, F), lambda i: (0, 0)),
        ],
        out_specs=pl.BlockSpec((bm, F), lambda i: (i, 0)),
        out_shape=jax.ShapeDtypeStruct((n, F), jnp.float32),
    )(y, g, w_n)


def kernel(inputs, w_s, w_n):
    n = inputs.shape[1]
    n_pad = -(-n // (NW * CH)) * (NW * CH)
    x, y, s0, s1, v0, v1 = _tc_prep(inputs, w_s, n_pad)
    g = _sc_gather_scale(x, s0, s1, v0, v1, n_pad)
    out = _tc_finish(y, g, w_n)
    return out[None]


# speedup vs baseline: 1.2213x; 1.1200x over previous
import functools

import jax
import jax.numpy as jnp
from jax import lax
from jax.experimental import pallas as pl
from jax.experimental.pallas import tpu as pltpu
from jax.experimental.pallas import tpu_sc as plsc

F = 128
FW = F + 2
NC, NS = 2, 16
NW = NC * NS
L = 16
CH = 80


def _sc_gather_scale(in2d, n_pad):
    n = in2d.shape[0]
    bw = n_pad // NW

    mesh = plsc.VectorSubcoreMesh(core_axis_name="c", subcore_axis_name="s")

    @functools.partial(
        pl.kernel,
        out_type=jax.ShapeDtypeStruct((n_pad, F), jnp.float32),
        mesh=mesh,
        compiler_params=pltpu.CompilerParams(needs_layout_passes=False),
        scratch_types=[
            pltpu.VMEM((CH,), jnp.int32),
            pltpu.VMEM((CH,), jnp.int32),
            pltpu.VMEM((CH,), jnp.float32),
            pltpu.VMEM((CH,), jnp.float32),
            pltpu.VMEM((CH, FW), jnp.float32),
            pltpu.VMEM((CH, F), jnp.float32),
            pltpu.VMEM((CH, F), jnp.float32),
            pltpu.VMEM((L, L), jnp.float32),
            pltpu.VMEM((L, L), jnp.float32),
            pltpu.VMEM((CH,), jnp.float32),
            pltpu.VMEM((CH,), jnp.float32),
            pltpu.VMEM_SHARED((n, F), jnp.float32),
            pltpu.SemaphoreType.DMA,
            pltpu.SemaphoreType.DMA,
            pltpu.SemaphoreType.DMA,
            pltpu.SemaphoreType.DMA,
        ],
    )
    def k(in_hbm, g_hbm,
          safe0, safe1, val0, val1, selfv, nb0, nb1, tr0, tr1,
          cbuf0, cbuf1, x_sh, sem_s, sem0, sem1, sem_sh):
        sid = lax.axis_index("s")
        wid = sid * NC + lax.axis_index("c")
        base = wid * bw

        rps = (n // NS) // 8 * 8
        rem = n - (NS - 1) * rps

        @pl.when(sid < NS - 1)
        def _():
            pltpu.async_copy(
                in_hbm.at[pl.ds(sid * rps, rps), pl.ds(0, F)],
                x_sh.at[pl.ds(sid * rps, rps)],
                sem_sh,
            )

        @pl.when(sid == NS - 1)
        def _():
            pltpu.async_copy(
                in_hbm.at[pl.ds((NS - 1) * rps, rem), pl.ds(0, F)],
                x_sh.at[pl.ds((NS - 1) * rps, rem)],
                sem_sh,
            )

        lane = lax.iota(jnp.int32, L)
        keep = lane >= 3

        @pl.when(sid < NS - 1)
        def _():
            pltpu.make_async_copy(
                in_hbm.at[pl.ds(sid * rps, rps), pl.ds(0, F)],
                x_sh.at[pl.ds(sid * rps, rps)],
                sem_sh,
            ).wait()

        @pl.when(sid == NS - 1)
        def _():
            pltpu.make_async_copy(
                in_hbm.at[pl.ds((NS - 1) * rps, rem), pl.ds(0, F)],
                x_sh.at[pl.ds((NS - 1) * rps, rem)],
                sem_sh,
            ).wait()

        plsc.subcore_barrier()

        for c in range(bw // CH):
            cbase = c * CH

            @pl.when(base + cbase < n)
            def _():
                pltpu.async_copy(
                    in_hbm.at[pl.ds(base + cbase, CH)], selfv, sem_s).wait()

                def mkidx(j, _):
                    av = j * L + lane
                    iv0 = plsc.load_gather(
                        selfv, [av, jnp.full((L,), F, jnp.int32)])
                    iv1 = plsc.load_gather(
                        selfv, [av, jnp.full((L,), F + 1, jnp.int32)])
                    i0 = iv0.astype(jnp.int32)
                    i1 = iv1.astype(jnp.int32)
                    valid0 = (i0 > 0) & (i0 < n)
                    valid1 = (i1 > 0) & (i1 < n)
                    self_idx = base + cbase + av
                    safe0[pl.ds(j * L, L)] = jnp.where(valid0, i0, self_idx)
                    safe1[pl.ds(j * L, L)] = jnp.where(valid1, i1, self_idx)
                    val0[pl.ds(j * L, L)] = jnp.where(valid0, 1.0, 0.0)
                    val1[pl.ds(j * L, L)] = jnp.where(valid1, 1.0, 0.0)
                    return 0

                lax.fori_loop(0, CH // L, mkidx, 0)

                cp0 = pltpu.async_copy(x_sh.at[safe0], nb0, sem0)
                cp1 = pltpu.async_copy(x_sh.at[safe1], nb1, sem1)
                cp0.wait()
                cp1.wait()

                def per_group(j, _):
                    gbase = j * L
                    for t in range(L):
                        a = gbase + t
                        acc0 = jnp.zeros((L,), jnp.float32)
                        acc1 = jnp.zeros((L,), jnp.float32)
                        for b in range(F // L):
                            s = selfv[a, pl.ds(b * L, L)]
                            e0 = s - nb0[a, pl.ds(b * L, L)]
                            e1 = s - nb1[a, pl.ds(b * L, L)]
                            if b == 0:
                                e0 = jnp.where(keep, e0, 0.0)
                                e1 = jnp.where(keep, e1, 0.0)
                            acc0 = acc0 + e0 * e0
                            acc1 = acc1 + e1 * e1
                        col = jnp.full((L,), t, jnp.int32)
                        plsc.store_scatter(tr0, [lane, col], acc0)
                        plsc.store_scatter(tr1, [lane, col], acc1)
                    d20 = jnp.zeros((L,), jnp.float32)
                    d21 = jnp.zeros((L,), jnp.float32)
                    for r in range(L):
                        d20 = d20 + tr0[r, :]
                        d21 = d21 + tr1[r, :]
                    c0 = jnp.where(
                        d20 > 0, jnp.where(d20 > 1e-6, 1.0 / d20, 1e6), 1.0)
                    c1 = jnp.where(
                        d21 > 0, jnp.where(d21 > 1e-6, 1.0 / d21, 1e6), 1.0)
                    cbuf0[pl.ds(gbase, L)] = c0 * val0[pl.ds(gbase, L)]
                    cbuf1[pl.ds(gbase, L)] = c1 * val1[pl.ds(gbase, L)]
                    return 0

                lax.fori_loop(0, CH // L, per_group, 0)

                def per_group_out(j, _):
                    gbase = j * L
                    cv0 = cbuf0[pl.ds(gbase, L)]
                    cv1 = cbuf1[pl.ds(gbase, L)]
                    for t in range(L):
                        a = gbase + t
                        c0 = cv0[t]
                        c1 = cv1[t]
                        for b in range(F // L):
                            nb0[a, pl.ds(b * L, L)] = (
                                c0 * nb0[a, pl.ds(b * L, L)]
                                + c1 * nb1[a, pl.ds(b * L, L)]
                            )
                    return 0

                lax.fori_loop(0, CH // L, per_group_out, 0)
                pltpu.sync_copy(nb0, g_hbm.at[pl.ds(base + cbase, CH)])

    return k(in2d)


def _tc_prep(inputs, w_s):
    n = inputs.shape[1]
    bm = 2000 if n % 2000 == 0 else 1000

    def body(in_ref, ws_ref, y_ref):
        x = in_ref[0][:, :F]
        y_ref[...] = jnp.dot(x, ws_ref[...], preferred_element_type=jnp.float32)

    return pl.pallas_call(
        body,
        grid=(n // bm,),
        in_specs=[
            pl.BlockSpec((1, bm, FW), lambda i: (0, i, 0)),
            pl.BlockSpec((F, F), lambda i: (0, 0)),
        ],
        out_specs=pl.BlockSpec((bm, F), lambda i: (i, 0)),
        out_shape=jax.ShapeDtypeStruct((n, F), jnp.float32),
    )(inputs, w_s)


def _tc_finish(y, g, w_n):
    n = y.shape[0]
    bm = 2000 if n % 2000 == 0 else 1000

    def body(y_ref, g_ref, wn_ref, o_ref):
        o_ref[...] = y_ref[...] + jnp.dot(
            g_ref[...], wn_ref[...], preferred_element_type=jnp.float32)

    return pl.pallas_call(
        body,
        grid=(n // bm,),
        in_specs=[
            pl.BlockSpec((bm, F), lambda i: (i, 0)),
            pl.BlockSpec((bm, F), lambda i: (i, 0)),
            pl.BlockSpec((F, F), lambda i: (0, 0)),
        ],
        out_specs=pl.BlockSpec((bm, F), lambda i: (i, 0)),
        out_shape=jax.ShapeDtypeStruct((n, F), jnp.float32),
    )(y, g, w_n)


def kernel(inputs, w_s, w_n):
    n = inputs.shape[1]
    n_pad = -(-n // (NW * CH)) * (NW * CH)
    in2d = inputs[0]
    g = _sc_gather_scale(in2d, n_pad)
    y = _tc_prep(inputs, w_s)
    out = _tc_finish(y, g, w_n)
    return out[None]
